# baseline (speedup 1.0000x reference)
import functools

import jax
import jax.numpy as jnp
import numpy as np
from jax import lax
from jax.experimental import pallas as pl
from jax.experimental.pallas import tpu as pltpu
from jax.experimental.pallas import tpu_sc as plsc

N = 150000
B = 2
C = 64
GY = GX = 512
GPSI = GR = 512
PLANE = GY * GX
NTEC = 16
NCORE = 2
ROWS = 74
PTS = ROWS * 128
NPAD = NTEC * PTS
SCRN = 1024
SCRB = 2 * PLANE
BUFW = 2 * PLANE + SCRN
SLICE = PLANE // NTEC
CH_PER_CORE = C // NCORE


def _idx_body(b_ref, y_ref, x_ref, src_ref, dst_ref, scr_ref, pid_ref):
    f = jnp.float32
    bi = b_ref[...]
    yi = y_ref[...]
    xi = x_ref[...]
    y = yi.astype(jnp.float32) * f(0.2) + f(-51.2)
    x = xi.astype(jnp.float32) * f(0.2) + f(-51.2)
    r = jnp.sqrt(x * x + y * y)
    xs = jnp.where(x == 0.0, f(1.0), x)
    at = jnp.arctan2(y / xs, jnp.ones_like(x))
    pi = f(np.pi)
    psi = jnp.where(
        x > 0, at,
        jnp.where((x == 0) & (y >= 0), f(np.pi / 2.0),
        jnp.where((x == 0) & (y < 0), f(-np.pi / 2.0),
        jnp.where(y >= 0, at + pi, at - pi))))
    rb = (r - f(0.0)) / f(0.142)
    pb = (psi - f(-np.pi)) / f(0.0123)
    ri = jnp.clip(rb.astype(jnp.int32), 0, GR - 1)
    pii = jnp.clip(pb.astype(jnp.int32), 0, GPSI - 1)
    r0 = lax.broadcasted_iota(jnp.int32, bi.shape, 0)
    c0 = lax.broadcasted_iota(jnp.int32, bi.shape, 1)
    flat = r0 * 128 + c0
    pad = SCRB + jnp.bitwise_and(flat, SCRN - 1)
    valid = bi < B
    src_ref[...] = jnp.where(valid, bi * PLANE + pii * GR + ri, pad)
    dst_ref[...] = jnp.where(valid, bi * PLANE + yi * GX + xi, pad)
    scr_ref[...] = pad
    pid_ref[...] = lax.bitcast_convert_type(flat, jnp.float32)


def _point_indices(voxel_coords):
    vc = jnp.pad(voxel_coords, ((0, NPAD - N), (0, 0)), constant_values=B)
    rows = NPAD // 128
    b2 = vc[:, 0].reshape(rows, 128)
    y2 = vc[:, 2].reshape(rows, 128)
    x2 = vc[:, 3].reshape(rows, 128)
    src, dst, scr, pid = pl.pallas_call(
        _idx_body,
        out_shape=[jax.ShapeDtypeStruct((rows, 128), jnp.int32)] * 3
        + [jax.ShapeDtypeStruct((rows, 128), jnp.float32)],
    )(b2, y2, x2)
    return (src.reshape(NTEC, ROWS, 128), dst.reshape(NTEC, ROWS, 128),
            scr.reshape(NTEC, ROWS, 128), pid.reshape(NTEC, ROWS, 128))


def _sc_body(sp_hbm, pv_hbm, srci_hbm, dsti_hbm, scri_hbm, pid_hbm, out_hbm,
             srci_v, dsti_v, scri_v, pid_v, dstf_v, pvv,
             sp_buf, pv_buf, sem_st, sem_g, sem_o):
    cid = lax.axis_index("c")
    sid = lax.axis_index("s")
    pltpu.sync_copy(srci_hbm.at[sid], srci_v)
    pltpu.sync_copy(dsti_hbm.at[sid], dsti_v)
    pltpu.sync_copy(scri_hbm.at[sid], scri_v)
    pltpu.sync_copy(pid_hbm.at[sid], pid_v)

    def id_scat(j, c):
        pltpu.async_copy(pid_v.at[j], pv_buf.at[dsti_v.at[j]], sem_g)
        return c
    lax.fori_loop(0, ROWS, id_scat, None)

    def id_scat_drain(j, c):
        pltpu.make_async_copy(pid_v.at[j], pv_buf.at[dsti_v.at[j]],
                              sem_g).wait()
        return c
    lax.fori_loop(0, ROWS, id_scat_drain, None)
    plsc.subcore_barrier()

    def id_gath(j, c):
        pltpu.async_copy(pv_buf.at[dsti_v.at[j]], pvv.at[j], sem_g)
        return c
    lax.fori_loop(0, ROWS, id_gath, None)

    def id_gath_drain(j, c):
        pltpu.make_async_copy(pv_buf.at[dsti_v.at[j]], pvv.at[j],
                              sem_g).wait()
        return c
    lax.fori_loop(0, ROWS, id_gath_drain, None)

    def sel(k, c):
        j = k // 8
        o = (k % 8) * 16
        s = pl.ds(o, 16)
        gi = lax.bitcast_convert_type(pvv[j, s], jnp.int32)
        pi = lax.bitcast_convert_type(pid_v[j, s], jnp.int32)
        dstf_v[j, s] = jnp.where(gi == pi, dsti_v[j, s], scri_v[j, s])
        return c
    lax.fori_loop(0, ROWS * 8, sel, None)
    plsc.subcore_barrier()

    off = sid * SLICE

    def fire_sp(ci):
        ch = cid * CH_PER_CORE + ci
        for b in range(B):
            pltpu.async_copy(
                sp_hbm.at[b, ch, pl.ds(off, SLICE)],
                sp_buf.at[pl.ds(b * PLANE + off, SLICE)], sem_st)

    def fire_pv(ci):
        ch = cid * CH_PER_CORE + ci
        for b in range(B):
            pltpu.async_copy(
                pv_hbm.at[b, ch, pl.ds(off, SLICE)],
                pv_buf.at[pl.ds(b * PLANE + off, SLICE)], sem_st)

    def wait_stage(ci):
        ch = cid * CH_PER_CORE + ci
        for b in range(B):
            pltpu.make_async_copy(
                sp_hbm.at[b, ch, pl.ds(off, SLICE)],
                sp_buf.at[pl.ds(b * PLANE + off, SLICE)], sem_st).wait()
            pltpu.make_async_copy(
                pv_hbm.at[b, ch, pl.ds(off, SLICE)],
                pv_buf.at[pl.ds(b * PLANE + off, SLICE)], sem_st).wait()

    fire_sp(0)
    fire_pv(0)

    def channel(ci, carry):
        wait_stage(ci)
        plsc.subcore_barrier()

        def gath_fire(j, c):
            pltpu.async_copy(pv_buf.at[srci_v.at[j]], pvv.at[j], sem_g)
            return c
        lax.fori_loop(0, ROWS, gath_fire, None)

        def gath_drain(j, c):
            pltpu.make_async_copy(pv_buf.at[srci_v.at[j]], pvv.at[j],
                                  sem_g).wait()
            return c
        lax.fori_loop(0, ROWS, gath_drain, None)

        def scat_fire(j, c):
            pltpu.async_copy(pvv.at[j], sp_buf.at[dstf_v.at[j]], sem_g,
                             add=True)
            return c
        lax.fori_loop(0, ROWS, scat_fire, None)

        def scat_drain(j, c):
            pltpu.make_async_copy(pvv.at[j], sp_buf.at[dstf_v.at[j]],
                                  sem_g).wait()
            return c
        lax.fori_loop(0, ROWS, scat_drain, None)
        plsc.subcore_barrier()

        nxt = jnp.minimum(ci + 1, CH_PER_CORE - 1)
        fire_pv(nxt)

        ch = cid * CH_PER_CORE + ci
        hs = []
        for b in range(B):
            hs.append(pltpu.async_copy(
                sp_buf.at[pl.ds(b * PLANE + off, SLICE)],
                out_hbm.at[b, ch, pl.ds(off, SLICE)], sem_o))
        for h in hs:
            h.wait()
        fire_sp(nxt)
        return carry

    lax.fori_loop(0, CH_PER_CORE, channel, None)
    wait_stage(CH_PER_CORE - 1)


_sc_scatter = functools.partial(
    pl.kernel,
    out_type=jax.ShapeDtypeStruct((B, C, PLANE), jnp.float32),
    mesh=plsc.VectorSubcoreMesh(core_axis_name="c", subcore_axis_name="s"),
    scratch_types=[
        pltpu.VMEM((ROWS, 128), jnp.int32),
        pltpu.VMEM((ROWS, 128), jnp.int32),
        pltpu.VMEM((ROWS, 128), jnp.int32),
        pltpu.VMEM((ROWS, 128), jnp.float32),
        pltpu.VMEM((ROWS, 128), jnp.int32),
        pltpu.VMEM((ROWS, 128), jnp.float32),
        pltpu.VMEM_SHARED((BUFW,), jnp.float32),
        pltpu.VMEM_SHARED((BUFW,), jnp.float32),
        pltpu.SemaphoreType.DMA,
        pltpu.SemaphoreType.DMA,
        pltpu.SemaphoreType.DMA,
    ],
)(_sc_body)


def kernel(voxel_coords, spatial_features, pview_spatial_features_0):
    src, dst, scr, pid = _point_indices(voxel_coords)
    sp2 = spatial_features.reshape(B, C, PLANE)
    pv2 = pview_spatial_features_0.reshape(B, C, PLANE)
    out2 = _sc_scatter(sp2, pv2, src, dst, scr, pid)
    return out2.reshape(B, C, GY, GX)


# speedup vs baseline: 1.2416x; 1.2416x over previous
import functools

import jax
import jax.numpy as jnp
import numpy as np
from jax import lax
from jax.experimental import pallas as pl
from jax.experimental.pallas import tpu as pltpu
from jax.experimental.pallas import tpu_sc as plsc

N = 150000
B = 2
C = 64
GY = GX = 512
GPSI = GR = 512
PLANE = GY * GX
NTEC = 16
NCORE = 2
ROWS = 74
PTS = ROWS * 128
NPAD = NTEC * PTS
SCRN = 1024
SCRB = 2 * PLANE
BUFW = 2 * PLANE + SCRN
SLICE = PLANE // NTEC
CH_PER_CORE = C // NCORE


def _idx_body(b_ref, y_ref, x_ref, src_ref, dst_ref, scr_ref, pid_ref):
    f = jnp.float32
    bi = b_ref[...]
    yi = y_ref[...]
    xi = x_ref[...]
    y = yi.astype(jnp.float32) * f(0.2) + f(-51.2)
    x = xi.astype(jnp.float32) * f(0.2) + f(-51.2)
    r = jnp.sqrt(x * x + y * y)
    xs = jnp.where(x == 0.0, f(1.0), x)
    at = jnp.arctan2(y / xs, jnp.ones_like(x))
    pi = f(np.pi)
    psi = jnp.where(
        x > 0, at,
        jnp.where((x == 0) & (y >= 0), f(np.pi / 2.0),
        jnp.where((x == 0) & (y < 0), f(-np.pi / 2.0),
        jnp.where(y >= 0, at + pi, at - pi))))
    rb = (r - f(0.0)) / f(0.142)
    pb = (psi - f(-np.pi)) / f(0.0123)
    ri = jnp.clip(rb.astype(jnp.int32), 0, GR - 1)
    pii = jnp.clip(pb.astype(jnp.int32), 0, GPSI - 1)
    r0 = lax.broadcasted_iota(jnp.int32, bi.shape, 0)
    c0 = lax.broadcasted_iota(jnp.int32, bi.shape, 1)
    flat = r0 * 128 + c0
    pad = SCRB + jnp.bitwise_and(flat, SCRN - 1)
    valid = bi < B
    src_ref[...] = jnp.where(valid, bi * PLANE + pii * GR + ri, pad)
    dst_ref[...] = jnp.where(valid, bi * PLANE + yi * GX + xi, pad)
    scr_ref[...] = pad
    pid_ref[...] = lax.bitcast_convert_type(flat, jnp.float32)


def _point_indices(voxel_coords):
    vc = jnp.pad(voxel_coords, ((0, NPAD - N), (0, 0)), constant_values=B)
    rows = NPAD // 128
    b2 = vc[:, 0].reshape(rows, 128)
    y2 = vc[:, 2].reshape(rows, 128)
    x2 = vc[:, 3].reshape(rows, 128)
    src, dst, scr, pid = pl.pallas_call(
        _idx_body,
        out_shape=[jax.ShapeDtypeStruct((rows, 128), jnp.int32)] * 3
        + [jax.ShapeDtypeStruct((rows, 128), jnp.float32)],
    )(b2, y2, x2)
    return (src.reshape(NTEC, ROWS, 128), dst.reshape(NTEC, ROWS, 128),
            scr.reshape(NTEC, ROWS, 128), pid.reshape(NTEC, ROWS, 128))


def _sc_body(sp_hbm, pv_hbm, srci_hbm, dsti_hbm, scri_hbm, pid_hbm, out_hbm,
             srci_v, dsti_v, scri_v, pid_v, dstf_v, pvv,
             sp_buf, pv_buf, sem_st, sem_g, sem_o):
    cid = lax.axis_index("c")
    sid = lax.axis_index("s")
    pltpu.sync_copy(srci_hbm.at[sid], srci_v)
    pltpu.sync_copy(dsti_hbm.at[sid], dsti_v)
    pltpu.sync_copy(scri_hbm.at[sid], scri_v)
    pltpu.sync_copy(pid_hbm.at[sid], pid_v)

    def id_scat(j, c):
        pltpu.async_copy(pid_v.at[j], pv_buf.at[dsti_v.at[j]], sem_g)
        return c
    lax.fori_loop(0, ROWS, id_scat, None)

    def id_scat_drain(j, c):
        pltpu.make_async_copy(pid_v.at[j], pv_buf.at[dsti_v.at[j]],
                              sem_g).wait()
        return c
    lax.fori_loop(0, ROWS, id_scat_drain, None)
    plsc.subcore_barrier()

    def id_gath(j, c):
        pltpu.async_copy(pv_buf.at[dsti_v.at[j]], pvv.at[j], sem_g)
        return c
    lax.fori_loop(0, ROWS, id_gath, None)

    def id_gath_drain(j, c):
        pltpu.make_async_copy(pv_buf.at[dsti_v.at[j]], pvv.at[j],
                              sem_g).wait()
        return c
    lax.fori_loop(0, ROWS, id_gath_drain, None)

    def sel(k, c):
        j = k // 8
        o = (k % 8) * 16
        s = pl.ds(o, 16)
        gi = lax.bitcast_convert_type(pvv[j, s], jnp.int32)
        pi = lax.bitcast_convert_type(pid_v[j, s], jnp.int32)
        dstf_v[j, s] = jnp.where(gi == pi, dsti_v[j, s], scri_v[j, s])
        return c
    lax.fori_loop(0, ROWS * 8, sel, None)
    plsc.subcore_barrier()

    off = sid * SLICE

    def fire_sp(ci):
        ch = cid * CH_PER_CORE + ci
        for b in range(B):
            pltpu.async_copy(
                sp_hbm.at[b * C + ch, pl.ds(off, SLICE)],
                sp_buf.at[pl.ds(b * PLANE + off, SLICE)], sem_st)

    def fire_pv(ci):
        ch = cid * CH_PER_CORE + ci
        for b in range(B):
            pltpu.async_copy(
                pv_hbm.at[b * C + ch, pl.ds(off, SLICE)],
                pv_buf.at[pl.ds(b * PLANE + off, SLICE)], sem_st)

    def wait_stage(ci):
        ch = cid * CH_PER_CORE + ci
        for b in range(B):
            pltpu.make_async_copy(
                sp_hbm.at[b * C + ch, pl.ds(off, SLICE)],
                sp_buf.at[pl.ds(b * PLANE + off, SLICE)], sem_st).wait()
            pltpu.make_async_copy(
                pv_hbm.at[b * C + ch, pl.ds(off, SLICE)],
                pv_buf.at[pl.ds(b * PLANE + off, SLICE)], sem_st).wait()

    fire_sp(0)
    fire_pv(0)

    def channel(ci, carry):
        wait_stage(ci)
        plsc.subcore_barrier()

        def gath_fire(j, c):
            pltpu.async_copy(pv_buf.at[srci_v.at[j]], pvv.at[j], sem_g)
            return c
        lax.fori_loop(0, ROWS, gath_fire, None)

        def gath_drain(j, c):
            pltpu.make_async_copy(pv_buf.at[srci_v.at[j]], pvv.at[j],
                                  sem_g).wait()
            return c
        lax.fori_loop(0, ROWS, gath_drain, None)

        def scat_fire(j, c):
            pltpu.async_copy(pvv.at[j], sp_buf.at[dstf_v.at[j]], sem_g,
                             add=True)
            return c
        lax.fori_loop(0, ROWS, scat_fire, None)

        def scat_drain(j, c):
            pltpu.make_async_copy(pvv.at[j], sp_buf.at[dstf_v.at[j]],
                                  sem_g).wait()
            return c
        lax.fori_loop(0, ROWS, scat_drain, None)
        plsc.subcore_barrier()

        nxt = jnp.minimum(ci + 1, CH_PER_CORE - 1)
        fire_pv(nxt)

        ch = cid * CH_PER_CORE + ci
        hs = []
        for b in range(B):
            row = b * C + ch
            hs.append(pltpu.async_copy(
                sp_buf.at[pl.ds(b * PLANE + off, SLICE)],
                out_hbm.at[row, pl.ds(off, SLICE)], sem_o))
        for h in hs:
            h.wait()
        fire_sp(nxt)
        return carry

    lax.fori_loop(0, CH_PER_CORE, channel, None)
    wait_stage(CH_PER_CORE - 1)


_sc_scatter = functools.partial(
    pl.kernel,
    out_type=jax.ShapeDtypeStruct((B * C, PLANE), jnp.float32),
    mesh=plsc.VectorSubcoreMesh(core_axis_name="c", subcore_axis_name="s"),
    scratch_types=[
        pltpu.VMEM((ROWS, 128), jnp.int32),
        pltpu.VMEM((ROWS, 128), jnp.int32),
        pltpu.VMEM((ROWS, 128), jnp.int32),
        pltpu.VMEM((ROWS, 128), jnp.float32),
        pltpu.VMEM((ROWS, 128), jnp.int32),
        pltpu.VMEM((ROWS, 128), jnp.float32),
        pltpu.VMEM_SHARED((BUFW,), jnp.float32),
        pltpu.VMEM_SHARED((BUFW,), jnp.float32),
        pltpu.SemaphoreType.DMA,
        pltpu.SemaphoreType.DMA,
        pltpu.SemaphoreType.DMA,
    ],
)(_sc_body)


def kernel(voxel_coords, spatial_features, pview_spatial_features_0):
    src, dst, scr, pid = _point_indices(voxel_coords)
    sp2 = spatial_features.reshape(B * C, PLANE)
    pv2 = pview_spatial_features_0.reshape(B * C, PLANE)
    out2 = _sc_scatter(sp2, pv2, src, dst, scr, pid)
    return out2.reshape(B, C, GY, GX)
